# baseline (speedup 1.0000x reference)
import functools

import jax
import jax.numpy as jnp
from jax import lax
from jax.experimental import pallas as pl
from jax.experimental.pallas import tpu as pltpu
from jax.experimental.pallas import tpu_sc as plsc

D = 128
CHUNK = 128
NBUF = 2
AHEAD = 1


def _body(x_hbm, idx_hbm, pos_hbm, table_hbm, out_hbm,
          idx_all, g_v, x_v, p_v, sem_g, sem_xp, sem_out,
          *, rows_per_worker, num_cores):
    wid = lax.axis_index("s") * num_cores + lax.axis_index("c")
    base = wid * rows_per_worker
    n_chunks = rows_per_worker // CHUNK

    pltpu.sync_copy(idx_hbm.at[pl.ds(base, rows_per_worker)], idx_all)

    def fire_in(s, k):
        row0 = base + k * CHUNK
        pltpu.async_copy(table_hbm.at[idx_all.at[pl.ds(k * CHUNK, CHUNK)]],
                         g_v.at[s], sem_g.at[s])
        pltpu.async_copy(x_hbm.at[pl.ds(row0, CHUNK)], x_v.at[s], sem_xp.at[s])
        pltpu.async_copy(pos_hbm.at[pl.ds(row0, CHUNK)], p_v.at[s], sem_xp.at[s])

    def wait_in(s, k):
        pltpu.make_async_copy(table_hbm.at[idx_all.at[pl.ds(k * CHUNK, CHUNK)]],
                              g_v.at[s], sem_g.at[s]).wait()
        row0 = base + k * CHUNK
        pltpu.make_async_copy(x_hbm.at[pl.ds(row0, CHUNK)], x_v.at[s],
                              sem_xp.at[s]).wait()
        pltpu.make_async_copy(pos_hbm.at[pl.ds(row0, CHUNK)], p_v.at[s],
                              sem_xp.at[s]).wait()

    def fire_out(s, k):
        row0 = base + k * CHUNK
        pltpu.async_copy(g_v.at[s], out_hbm.at[pl.ds(row0, CHUNK)],
                         sem_out.at[s])

    def wait_out(s, k):
        row0 = base + k * CHUNK
        pltpu.make_async_copy(g_v.at[s], out_hbm.at[pl.ds(row0, CHUNK)],
                              sem_out.at[s]).wait()

    def compute(s):
        def vec_body(i, carry):
            r = i >> 3
            c = (i & 7) * 16
            sl = pl.ds(c, 16)
            plsc.addupdate(g_v.at[s, r, sl], x_v[s, r, sl] + p_v[s, r, sl])
            return carry

        lax.fori_loop(0, CHUNK * (D // 16), vec_body, 0, unroll=8)

    for k in range(AHEAD):
        fire_in(k % NBUF, k)

    def outer(k0, carry):
        for s in range(NBUF):
            k = k0 * NBUF + s
            t = (s + AHEAD) % NBUF

            @pl.when(k + AHEAD < n_chunks)
            def _():
                @pl.when(k + AHEAD >= NBUF)
                def _():
                    wait_out(t, k + AHEAD - NBUF)
                fire_in(t, k + AHEAD)

            wait_in(s, k)
            compute(s)
            fire_out(s, k)
        return carry

    lax.fori_loop(0, n_chunks // NBUF, outer, 0)

    for s in range(NBUF):
        wait_out(s, n_chunks - NBUF + s)


def kernel(x, variable, pos_emb, variable_table):
    B, S, d = x.shape
    n = B * S
    xf = x.reshape(n, d)
    pf = pos_emb.reshape(n, d)
    idx = variable.reshape(n).astype(jnp.int32)

    info = plsc.get_sparse_core_info()
    nw = info.num_cores * info.num_subcores
    rows_per_worker = n // nw
    assert rows_per_worker % (CHUNK * NBUF) == 0

    mesh = plsc.VectorSubcoreMesh(core_axis_name="c", subcore_axis_name="s")
    run = pl.kernel(
        functools.partial(_body, rows_per_worker=rows_per_worker,
                          num_cores=info.num_cores),
        out_type=jax.ShapeDtypeStruct((n, d), jnp.float32),
        mesh=mesh,
        scratch_types=[
            pltpu.VMEM((rows_per_worker,), jnp.int32),
            pltpu.VMEM((NBUF, CHUNK, D), jnp.float32),
            pltpu.VMEM((NBUF, CHUNK, D), jnp.float32),
            pltpu.VMEM((NBUF, CHUNK, D), jnp.float32),
            pltpu.SemaphoreType.DMA((NBUF,)),
            pltpu.SemaphoreType.DMA((NBUF,)),
            pltpu.SemaphoreType.DMA((NBUF,)),
        ],
    )
    out = run(xf, idx, pf, variable_table)
    return out.reshape(B, S, d)


# speedup vs baseline: 1.0166x; 1.0166x over previous
import functools

import jax
import jax.numpy as jnp
from jax import lax
from jax.experimental import pallas as pl
from jax.experimental.pallas import tpu as pltpu
from jax.experimental.pallas import tpu_sc as plsc

D = 128
CHUNK = 32
NBUF = 10
AHEAD = 5


def _body(x_hbm, idx_hbm, pos_hbm, table_hbm, out_hbm,
          idx_all, g_v, x_v, p_v, sem_g, sem_xp, sem_out,
          *, rows_per_worker, num_cores):
    wid = lax.axis_index("s") * num_cores + lax.axis_index("c")
    base = wid * rows_per_worker
    n_chunks = rows_per_worker // CHUNK

    pltpu.sync_copy(idx_hbm.at[pl.ds(base, rows_per_worker)], idx_all)

    def fire_in(s, k):
        row0 = base + k * CHUNK
        pltpu.async_copy(table_hbm.at[idx_all.at[pl.ds(k * CHUNK, CHUNK)]],
                         g_v.at[s], sem_g.at[s])
        pltpu.async_copy(x_hbm.at[pl.ds(row0, CHUNK)], x_v.at[s], sem_xp.at[s])
        pltpu.async_copy(pos_hbm.at[pl.ds(row0, CHUNK)], p_v.at[s], sem_xp.at[s])

    def wait_in(s, k):
        pltpu.make_async_copy(table_hbm.at[idx_all.at[pl.ds(k * CHUNK, CHUNK)]],
                              g_v.at[s], sem_g.at[s]).wait()
        row0 = base + k * CHUNK
        pltpu.make_async_copy(x_hbm.at[pl.ds(row0, CHUNK)], x_v.at[s],
                              sem_xp.at[s]).wait()
        pltpu.make_async_copy(pos_hbm.at[pl.ds(row0, CHUNK)], p_v.at[s],
                              sem_xp.at[s]).wait()

    def fire_out(s, k):
        row0 = base + k * CHUNK
        pltpu.async_copy(g_v.at[s], out_hbm.at[pl.ds(row0, CHUNK)],
                         sem_out.at[s])

    def wait_out(s, k):
        row0 = base + k * CHUNK
        pltpu.make_async_copy(g_v.at[s], out_hbm.at[pl.ds(row0, CHUNK)],
                              sem_out.at[s]).wait()

    def compute(s):
        def vec_body(i, carry):
            r = i >> 3
            c = (i & 7) * 16
            sl = pl.ds(c, 16)
            plsc.addupdate(g_v.at[s, r, sl], x_v[s, r, sl] + p_v[s, r, sl])
            return carry

        lax.fori_loop(0, CHUNK * (D // 16), vec_body, 0, unroll=8)

    for k in range(AHEAD):
        fire_in(k % NBUF, k)

    def outer(k0, carry):
        for s in range(NBUF):
            k = k0 * NBUF + s
            t = (s + AHEAD) % NBUF

            @pl.when(k + AHEAD < n_chunks)
            def _():
                @pl.when(k + AHEAD >= NBUF)
                def _():
                    wait_out(t, k + AHEAD - NBUF)
                fire_in(t, k + AHEAD)

            wait_in(s, k)
            compute(s)
            fire_out(s, k)
        return carry

    lax.fori_loop(0, n_chunks // NBUF, outer, 0)

    for s in range(NBUF):
        wait_out(s, n_chunks - NBUF + s)


def kernel(x, variable, pos_emb, variable_table):
    B, S, d = x.shape
    n = B * S
    xf = x.reshape(n, d)
    pf = pos_emb.reshape(n, d)
    idx = variable.reshape(n).astype(jnp.int32)

    info = plsc.get_sparse_core_info()
    nw = info.num_cores * info.num_subcores
    rows_per_worker = n // nw
    assert rows_per_worker % (CHUNK * NBUF) == 0

    mesh = plsc.VectorSubcoreMesh(core_axis_name="c", subcore_axis_name="s")
    run = pl.kernel(
        functools.partial(_body, rows_per_worker=rows_per_worker,
                          num_cores=info.num_cores),
        out_type=jax.ShapeDtypeStruct((n, d), jnp.float32),
        mesh=mesh,
        scratch_types=[
            pltpu.VMEM((rows_per_worker,), jnp.int32),
            pltpu.VMEM((NBUF, CHUNK, D), jnp.float32),
            pltpu.VMEM((NBUF, CHUNK, D), jnp.float32),
            pltpu.VMEM((NBUF, CHUNK, D), jnp.float32),
            pltpu.SemaphoreType.DMA((NBUF,)),
            pltpu.SemaphoreType.DMA((NBUF,)),
            pltpu.SemaphoreType.DMA((NBUF,)),
        ],
    )
    out = run(xf, idx, pf, variable_table)
    return out.reshape(B, S, d)


# speedup vs baseline: 1.0259x; 1.0092x over previous
import functools

import jax
import jax.numpy as jnp
from jax import lax
from jax.experimental import pallas as pl
from jax.experimental.pallas import tpu as pltpu
from jax.experimental.pallas import tpu_sc as plsc

D = 128
CHUNK = 32
NBUF = 8
AHEAD = 4


def _body(x_hbm, idx_hbm, pos_hbm, table_hbm, out_hbm,
          idx_all, g_v, x_v, p_v, sem_g, sem_xp, sem_out,
          *, rows_per_worker, num_cores):
    wid = lax.axis_index("s") * num_cores + lax.axis_index("c")
    base = wid * rows_per_worker
    n_chunks = rows_per_worker // CHUNK

    pltpu.sync_copy(idx_hbm.at[pl.ds(base, rows_per_worker)], idx_all)

    def fire_in(s, k):
        row0 = base + k * CHUNK
        pltpu.async_copy(table_hbm.at[idx_all.at[pl.ds(k * CHUNK, CHUNK)]],
                         g_v.at[s], sem_g.at[s])
        pltpu.async_copy(x_hbm.at[pl.ds(row0, CHUNK)], x_v.at[s], sem_xp.at[s])
        pltpu.async_copy(pos_hbm.at[pl.ds(row0, CHUNK)], p_v.at[s], sem_xp.at[s])

    def wait_in(s, k):
        pltpu.make_async_copy(table_hbm.at[idx_all.at[pl.ds(k * CHUNK, CHUNK)]],
                              g_v.at[s], sem_g.at[s]).wait()
        row0 = base + k * CHUNK
        pltpu.make_async_copy(x_hbm.at[pl.ds(row0, CHUNK)], x_v.at[s],
                              sem_xp.at[s]).wait()
        pltpu.make_async_copy(pos_hbm.at[pl.ds(row0, CHUNK)], p_v.at[s],
                              sem_xp.at[s]).wait()

    def fire_out(s, k):
        row0 = base + k * CHUNK
        pltpu.async_copy(g_v.at[s], out_hbm.at[pl.ds(row0, CHUNK)],
                         sem_out.at[s])

    def wait_out(s, k):
        row0 = base + k * CHUNK
        pltpu.make_async_copy(g_v.at[s], out_hbm.at[pl.ds(row0, CHUNK)],
                              sem_out.at[s]).wait()

    def compute(s):
        def vec_body(i, carry):
            r = i >> 3
            c = (i & 7) * 16
            sl = pl.ds(c, 16)
            plsc.addupdate(g_v.at[s, r, sl], x_v[s, r, sl] + p_v[s, r, sl])
            return carry

        lax.fori_loop(0, CHUNK * (D // 16), vec_body, 0, unroll=8)

    for k in range(AHEAD):
        fire_in(k % NBUF, k)

    def outer(k0, carry):
        for s in range(NBUF):
            k = k0 * NBUF + s
            t = (s + AHEAD) % NBUF

            @pl.when(k + AHEAD < n_chunks)
            def _():
                @pl.when(k + AHEAD >= NBUF)
                def _():
                    wait_out(t, k + AHEAD - NBUF)
                fire_in(t, k + AHEAD)

            wait_in(s, k)
            compute(s)
            fire_out(s, k)
        return carry

    lax.fori_loop(0, n_chunks // NBUF, outer, 0)

    for s in range(NBUF):
        wait_out(s, n_chunks - NBUF + s)


def kernel(x, variable, pos_emb, variable_table):
    B, S, d = x.shape
    n = B * S
    xf = x.reshape(n, d)
    pf = pos_emb.reshape(n, d)
    idx = variable.reshape(n).astype(jnp.int32)

    info = plsc.get_sparse_core_info()
    nw = info.num_cores * info.num_subcores
    rows_per_worker = n // nw
    assert rows_per_worker % (CHUNK * NBUF) == 0

    mesh = plsc.VectorSubcoreMesh(core_axis_name="c", subcore_axis_name="s")
    run = pl.kernel(
        functools.partial(_body, rows_per_worker=rows_per_worker,
                          num_cores=info.num_cores),
        out_type=jax.ShapeDtypeStruct((n, d), jnp.float32),
        mesh=mesh,
        scratch_types=[
            pltpu.VMEM((rows_per_worker,), jnp.int32),
            pltpu.VMEM((NBUF, CHUNK, D), jnp.float32),
            pltpu.VMEM((NBUF, CHUNK, D), jnp.float32),
            pltpu.VMEM((NBUF, CHUNK, D), jnp.float32),
            pltpu.SemaphoreType.DMA((NBUF,)),
            pltpu.SemaphoreType.DMA((NBUF,)),
            pltpu.SemaphoreType.DMA((NBUF,)),
        ],
    )
    out = run(xf, idx, pf, variable_table)
    return out.reshape(B, S, d)


# speedup vs baseline: 1.0269x; 1.0010x over previous
import functools

import jax
import jax.numpy as jnp
from jax import lax
from jax.experimental import pallas as pl
from jax.experimental.pallas import tpu as pltpu
from jax.experimental.pallas import tpu_sc as plsc

D = 128
CHUNK = 32
NBUF = 8
AHEAD = 4


def _body(x_hbm, idx_hbm, pos_hbm, table_hbm, out_hbm,
          idx_all, g_v, x_v, p_v, sem_g, sem_xp, sem_out,
          *, rows_per_worker, num_cores):
    wid = lax.axis_index("s") * num_cores + lax.axis_index("c")
    base = wid * rows_per_worker
    n_chunks = rows_per_worker // CHUNK

    pltpu.sync_copy(idx_hbm.at[pl.ds(base, rows_per_worker)], idx_all)

    def fire_in(s, k):
        row0 = base + k * CHUNK
        pltpu.async_copy(x_hbm.at[pl.ds(row0, CHUNK)], x_v.at[s], sem_xp.at[s])
        pltpu.async_copy(pos_hbm.at[pl.ds(row0, CHUNK)], p_v.at[s], sem_xp.at[s])
        pltpu.async_copy(table_hbm.at[idx_all.at[pl.ds(k * CHUNK, CHUNK)]],
                         g_v.at[s], sem_g.at[s])

    def wait_in(s, k):
        pltpu.make_async_copy(table_hbm.at[idx_all.at[pl.ds(k * CHUNK, CHUNK)]],
                              g_v.at[s], sem_g.at[s]).wait()
        row0 = base + k * CHUNK
        pltpu.make_async_copy(x_hbm.at[pl.ds(row0, CHUNK)], x_v.at[s],
                              sem_xp.at[s]).wait()
        pltpu.make_async_copy(pos_hbm.at[pl.ds(row0, CHUNK)], p_v.at[s],
                              sem_xp.at[s]).wait()

    def fire_out(s, k):
        row0 = base + k * CHUNK
        pltpu.async_copy(g_v.at[s], out_hbm.at[pl.ds(row0, CHUNK)],
                         sem_out.at[s])

    def wait_out(s, k):
        row0 = base + k * CHUNK
        pltpu.make_async_copy(g_v.at[s], out_hbm.at[pl.ds(row0, CHUNK)],
                              sem_out.at[s]).wait()

    def compute(s):
        def vec_body(i, carry):
            r = i >> 3
            c = (i & 7) * 16
            sl = pl.ds(c, 16)
            plsc.addupdate(g_v.at[s, r, sl], x_v[s, r, sl] + p_v[s, r, sl])
            return carry

        lax.fori_loop(0, CHUNK * (D // 16), vec_body, 0, unroll=8)

    for k in range(AHEAD):
        fire_in(k % NBUF, k)

    def outer(k0, carry):
        for s in range(NBUF):
            k = k0 * NBUF + s
            t = (s + AHEAD) % NBUF

            @pl.when(k + AHEAD < n_chunks)
            def _():
                @pl.when(k + AHEAD >= NBUF)
                def _():
                    wait_out(t, k + AHEAD - NBUF)
                fire_in(t, k + AHEAD)

            wait_in(s, k)
            compute(s)
            fire_out(s, k)
        return carry

    lax.fori_loop(0, n_chunks // NBUF, outer, 0)

    for s in range(NBUF):
        wait_out(s, n_chunks - NBUF + s)


def kernel(x, variable, pos_emb, variable_table):
    B, S, d = x.shape
    n = B * S
    xf = x.reshape(n, d)
    pf = pos_emb.reshape(n, d)
    idx = variable.reshape(n).astype(jnp.int32)

    info = plsc.get_sparse_core_info()
    nw = info.num_cores * info.num_subcores
    rows_per_worker = n // nw
    assert rows_per_worker % (CHUNK * NBUF) == 0

    mesh = plsc.VectorSubcoreMesh(core_axis_name="c", subcore_axis_name="s")
    run = pl.kernel(
        functools.partial(_body, rows_per_worker=rows_per_worker,
                          num_cores=info.num_cores),
        out_type=jax.ShapeDtypeStruct((n, d), jnp.float32),
        mesh=mesh,
        scratch_types=[
            pltpu.VMEM((rows_per_worker,), jnp.int32),
            pltpu.VMEM((NBUF, CHUNK, D), jnp.float32),
            pltpu.VMEM((NBUF, CHUNK, D), jnp.float32),
            pltpu.VMEM((NBUF, CHUNK, D), jnp.float32),
            pltpu.SemaphoreType.DMA((NBUF,)),
            pltpu.SemaphoreType.DMA((NBUF,)),
            pltpu.SemaphoreType.DMA((NBUF,)),
        ],
    )
    out = run(xf, idx, pf, variable_table)
    return out.reshape(B, S, d)
